# baseline (speedup 1.0000x reference)
import functools

import jax
import jax.numpy as jnp
from jax.experimental import pallas as pl
from jax.experimental.pallas import tpu as pltpu
from jax.experimental.pallas import tpu_sc as plsc

_MARGIN = 0.0
_MOMENTUM = 0.95
_MAX_S = 20.0
_PREV_S = 20.0
_RUNNING_B = 1000.0
_RUNNING_COS = 0.7

_BR = 32
_LANES = 128
_SC_COLS = 25632


def _pass1_kernel(y_sm, x_ref, *args):
    seg_refs = args[:_BR]
    sums_ref, tgt_ref = args[_BR], args[_BR + 1]
    i = pl.program_id(0)
    x = x_ref[...]
    sums_ref[...] = jnp.sum(jnp.exp(x * _PREV_S), axis=1).reshape(1, 1, _BR)
    parts = []
    for r in range(_BR):
        lane = y_sm[i * _BR + r] % _LANES
        m = jax.lax.broadcasted_iota(jnp.int32, (1, _LANES), 1) == lane
        parts.append(jnp.where(m, seg_refs[r][pl.ds(r, 1), :], 0.0))
    tgt = jnp.sum(jnp.concatenate(parts, axis=0), axis=1)
    tgt_ref[...] = tgt.reshape(1, 1, _BR)


def _pass2_kernel(s_ref, x_ref, sums_ref):
    x = x_ref[...]
    sums_ref[...] = jnp.sum(jnp.exp(x * s_ref[0]), axis=1).reshape(1, 1, _BR)


def _make_sc_rowsum(B, C):
    info = plsc.get_sparse_core_info()
    nc, ns = info.num_cores, info.num_subcores
    nw = nc * ns
    rpw = B // nw
    c0 = C - _SC_COLS
    n16 = _SC_COLS // 16
    mesh = plsc.VectorSubcoreMesh(core_axis_name="c", subcore_axis_name="s")

    @functools.partial(
        pl.kernel,
        mesh=mesh,
        out_type=jax.ShapeDtypeStruct((B, 16), jnp.float32),
        scratch_types=[
            pltpu.VMEM((_SC_COLS,), jnp.float32),
            pltpu.VMEM((rpw, 16), jnp.float32),
        ],
    )
    def sc_rowsum(x_hbm, out_hbm, buf, out_v):
        wid = jax.lax.axis_index("s") * nc + jax.lax.axis_index("c")
        base = wid * rpw

        def row_body(j, carry):
            pltpu.sync_copy(x_hbm.at[base + j, pl.ds(c0, _SC_COLS)], buf)

            def chunk(k, accs):
                a0, a1, a2, a3 = accs
                b = k * 64
                a0 = a0 + jnp.exp(buf[pl.ds(b, 16)] * _PREV_S)
                a1 = a1 + jnp.exp(buf[pl.ds(b + 16, 16)] * _PREV_S)
                a2 = a2 + jnp.exp(buf[pl.ds(b + 32, 16)] * _PREV_S)
                a3 = a3 + jnp.exp(buf[pl.ds(b + 48, 16)] * _PREV_S)
                return (a0, a1, a2, a3)

            z = jnp.zeros((16,), jnp.float32)
            a0, a1, a2, a3 = jax.lax.fori_loop(
                0, n16 // 4, chunk, (z, z, z, z)
            )
            out_v[j, :] = (a0 + a1) + (a2 + a3)
            return carry

        jax.lax.fori_loop(0, rpw, row_body, 0)
        pltpu.sync_copy(out_v, out_hbm.at[pl.ds(base, rpw)])

    return sc_rowsum


def kernel(cosine, y_true):
    B, C = cosine.shape
    y_true = y_true.astype(jnp.int32)
    nb = B // _BR
    c_tc = C - _SC_COLS

    sc_sums = _make_sc_rowsum(B, C)(cosine)

    def _seg_spec(r):
        return pl.BlockSpec(
            (_BR, _LANES), lambda i, y: (i, y[i * _BR + r] // _LANES)
        )

    grid_spec = pltpu.PrefetchScalarGridSpec(
        num_scalar_prefetch=1,
        grid=(nb,),
        in_specs=[
            pl.BlockSpec((_BR, c_tc), lambda i, y: (i, 0)),
            *[_seg_spec(r) for r in range(_BR)],
        ],
        out_specs=[
            pl.BlockSpec((1, 1, _BR), lambda i, y: (i, 0, 0)),
            pl.BlockSpec((1, 1, _BR), lambda i, y: (i, 0, 0)),
        ],
    )
    sums3, tgt3 = pl.pallas_call(
        _pass1_kernel,
        grid_spec=grid_spec,
        out_shape=[
            jax.ShapeDtypeStruct((nb, 1, _BR), jnp.float32),
            jax.ShapeDtypeStruct((nb, 1, _BR), jnp.float32),
        ],
        compiler_params=pltpu.CompilerParams(
            dimension_semantics=("parallel",)
        ),
    )(y_true, cosine, *([cosine] * _BR))
    sums = sums3.reshape(B) + jnp.sum(sc_sums, axis=1)
    tgt = tgt3.reshape(B)

    total = jnp.sum(sums)
    b_batch = (total - jnp.sum(jnp.exp(tgt * _PREV_S))) / B
    med_cos = jnp.median(tgt)
    running_b = _RUNNING_B * _MOMENTUM + b_batch * (1.0 - _MOMENTUM)
    running_cos = _RUNNING_COS * _MOMENTUM + med_cos * (1.0 - _MOMENTUM)
    prev_s = jnp.log(running_b) / (jnp.maximum(running_cos, 0.7) - _MARGIN)
    prev_s = jnp.minimum(prev_s, _MAX_S)

    def _fast(_):
        return jnp.mean(jnp.log(sums) - prev_s * tgt)

    def _slow(_):
        sums2 = pl.pallas_call(
            _pass2_kernel,
            grid=(nb,),
            in_specs=[
                pl.BlockSpec(memory_space=pltpu.SMEM),
                pl.BlockSpec((_BR, C), lambda i: (i, 0)),
            ],
            out_specs=pl.BlockSpec((1, 1, _BR), lambda i: (i, 0, 0)),
            out_shape=jax.ShapeDtypeStruct((nb, 1, _BR), jnp.float32),
        )(prev_s[None], cosine)
        return jnp.mean(jnp.log(sums2.reshape(B)) - prev_s * tgt)

    return jax.lax.cond(prev_s == _PREV_S, _fast, _slow, None)


# speedup vs baseline: 1.0147x; 1.0147x over previous
import functools

import jax
import jax.numpy as jnp
from jax.experimental import pallas as pl
from jax.experimental.pallas import tpu as pltpu
from jax.experimental.pallas import tpu_sc as plsc

_MARGIN = 0.0
_MOMENTUM = 0.95
_MAX_S = 20.0
_PREV_S = 20.0
_RUNNING_B = 1000.0
_RUNNING_COS = 0.7

_BR = 32
_LANES = 128
_SC_COLS = 12832


def _pass1_kernel(y_sm, x_ref, *args):
    seg_refs = args[:_BR]
    sums_ref, tgt_ref = args[_BR], args[_BR + 1]
    i = pl.program_id(0)
    x = x_ref[...]
    sums_ref[...] = jnp.sum(jnp.exp(x * _PREV_S), axis=1).reshape(1, 1, _BR)
    parts = []
    for r in range(_BR):
        lane = y_sm[i * _BR + r] % _LANES
        m = jax.lax.broadcasted_iota(jnp.int32, (1, _LANES), 1) == lane
        parts.append(jnp.where(m, seg_refs[r][pl.ds(r, 1), :], 0.0))
    tgt = jnp.sum(jnp.concatenate(parts, axis=0), axis=1)
    tgt_ref[...] = tgt.reshape(1, 1, _BR)


def _pass2_kernel(s_ref, x_ref, sums_ref):
    x = x_ref[...]
    sums_ref[...] = jnp.sum(jnp.exp(x * s_ref[0]), axis=1).reshape(1, 1, _BR)


def _make_sc_rowsum(B, C):
    info = plsc.get_sparse_core_info()
    nc, ns = info.num_cores, info.num_subcores
    nw = nc * ns
    rpw = B // nw
    c0 = C - _SC_COLS
    n16 = _SC_COLS // 16
    mesh = plsc.VectorSubcoreMesh(core_axis_name="c", subcore_axis_name="s")

    @functools.partial(
        pl.kernel,
        mesh=mesh,
        out_type=jax.ShapeDtypeStruct((B, 16), jnp.float32),
        scratch_types=[
            pltpu.VMEM((_SC_COLS,), jnp.float32),
            pltpu.VMEM((rpw, 16), jnp.float32),
        ],
    )
    def sc_rowsum(x_hbm, out_hbm, buf, out_v):
        wid = jax.lax.axis_index("s") * nc + jax.lax.axis_index("c")
        base = wid * rpw

        def row_body(j, carry):
            pltpu.sync_copy(x_hbm.at[base + j, pl.ds(c0, _SC_COLS)], buf)

            def chunk(k, accs):
                a0, a1, a2, a3 = accs
                b = k * 64
                a0 = a0 + jnp.exp(buf[pl.ds(b, 16)] * _PREV_S)
                a1 = a1 + jnp.exp(buf[pl.ds(b + 16, 16)] * _PREV_S)
                a2 = a2 + jnp.exp(buf[pl.ds(b + 32, 16)] * _PREV_S)
                a3 = a3 + jnp.exp(buf[pl.ds(b + 48, 16)] * _PREV_S)
                return (a0, a1, a2, a3)

            z = jnp.zeros((16,), jnp.float32)
            a0, a1, a2, a3 = jax.lax.fori_loop(
                0, n16 // 4, chunk, (z, z, z, z)
            )
            out_v[j, :] = (a0 + a1) + (a2 + a3)
            return carry

        jax.lax.fori_loop(0, rpw, row_body, 0)
        pltpu.sync_copy(out_v, out_hbm.at[pl.ds(base, rpw)])

    return sc_rowsum


def kernel(cosine, y_true):
    B, C = cosine.shape
    y_true = y_true.astype(jnp.int32)
    nb = B // _BR
    c_tc = C - _SC_COLS

    sc_sums = _make_sc_rowsum(B, C)(cosine)

    def _seg_spec(r):
        return pl.BlockSpec(
            (_BR, _LANES), lambda i, y: (i, y[i * _BR + r] // _LANES)
        )

    grid_spec = pltpu.PrefetchScalarGridSpec(
        num_scalar_prefetch=1,
        grid=(nb,),
        in_specs=[
            pl.BlockSpec((_BR, c_tc), lambda i, y: (i, 0)),
            *[_seg_spec(r) for r in range(_BR)],
        ],
        out_specs=[
            pl.BlockSpec((1, 1, _BR), lambda i, y: (i, 0, 0)),
            pl.BlockSpec((1, 1, _BR), lambda i, y: (i, 0, 0)),
        ],
    )
    sums3, tgt3 = pl.pallas_call(
        _pass1_kernel,
        grid_spec=grid_spec,
        out_shape=[
            jax.ShapeDtypeStruct((nb, 1, _BR), jnp.float32),
            jax.ShapeDtypeStruct((nb, 1, _BR), jnp.float32),
        ],
        compiler_params=pltpu.CompilerParams(
            dimension_semantics=("parallel",)
        ),
    )(y_true, cosine, *([cosine] * _BR))
    sums = sums3.reshape(B) + jnp.sum(sc_sums, axis=1)
    tgt = tgt3.reshape(B)

    total = jnp.sum(sums)
    b_batch = (total - jnp.sum(jnp.exp(tgt * _PREV_S))) / B
    med_cos = jnp.median(tgt)
    running_b = _RUNNING_B * _MOMENTUM + b_batch * (1.0 - _MOMENTUM)
    running_cos = _RUNNING_COS * _MOMENTUM + med_cos * (1.0 - _MOMENTUM)
    prev_s = jnp.log(running_b) / (jnp.maximum(running_cos, 0.7) - _MARGIN)
    prev_s = jnp.minimum(prev_s, _MAX_S)

    def _fast(_):
        return jnp.mean(jnp.log(sums) - prev_s * tgt)

    def _slow(_):
        sums2 = pl.pallas_call(
            _pass2_kernel,
            grid=(nb,),
            in_specs=[
                pl.BlockSpec(memory_space=pltpu.SMEM),
                pl.BlockSpec((_BR, C), lambda i: (i, 0)),
            ],
            out_specs=pl.BlockSpec((1, 1, _BR), lambda i: (i, 0, 0)),
            out_shape=jax.ShapeDtypeStruct((nb, 1, _BR), jnp.float32),
        )(prev_s[None], cosine)
        return jnp.mean(jnp.log(sums2.reshape(B)) - prev_s * tgt)

    return jax.lax.cond(prev_s == _PREV_S, _fast, _slow, None)
